# baseline (speedup 1.0000x reference)
import functools

import jax
import jax.numpy as jnp
from jax import lax
from jax.experimental import pallas as pl
from jax.experimental.pallas import tpu as pltpu
from jax.experimental.pallas import tpu_sc as plsc

T = 2048
D = 1024
E = 8
NGROUP = 4
I = 512
NSH = 2
SCALE = 1.5

TB = 256
NT = T // TB
C = T
CB = C // TB
SLOTS = E * C
SB = 512
CBS = C // SB
NB = 16

_NC, _NS = 2, 16
_NW = _NC * _NS
TPW = T // _NW


def _dot_t(a, b):
    return jax.lax.dot_general(a, b, (((1,), (1,)), ((), ())),
                               preferred_element_type=jnp.float32)


def _router_body(x_ref, gw_ref, bias_ref, d0_ref, d1_ref, w1_ref,
                 w2_ref, cnt_ref, bexp_ref, brow_ref):
    x = x_ref[...]
    logits = _dot_t(x, gw_ref[...])
    m = jnp.max(logits, axis=1, keepdims=True)
    ex = jnp.exp(logits - m)
    p = ex / jnp.sum(ex, axis=1, keepdims=True)

    li = jax.lax.broadcasted_iota(jnp.int32, (T, E), 1)
    pcol = [jnp.sum(jnp.where(li == e, p, 0.0), axis=1, keepdims=True)
            for e in range(E)]
    sccol = [pcol[e] + bias_ref[0, e] for e in range(E)]

    gs = [sccol[2 * g] + sccol[2 * g + 1] for g in range(NGROUP)]

    neg = jnp.float32(-jnp.inf)

    def first_argmax(cols):
        v = cols[0]
        for c in cols[1:]:
            v = jnp.maximum(v, c)
        idx = jnp.full_like(v, float(len(cols)))
        for j in range(len(cols) - 1, -1, -1):
            idx = jnp.where(cols[j] == v, float(j), idx)
        return v, idx

    _, g1 = first_argmax(gs)
    gs2 = [jnp.where(g1 == float(g), neg, gs[g]) for g in range(NGROUP)]
    _, g2 = first_argmax(gs2)

    masked = [jnp.where((g1 == float(e // 2)) | (g2 == float(e // 2)),
                        sccol[e], neg) for e in range(E)]
    _, e1 = first_argmax(masked)
    masked2 = [jnp.where(e1 == float(e), neg, masked[e]) for e in range(E)]
    _, e2 = first_argmax(masked2)

    def pick(idx, cols):
        v = jnp.zeros_like(cols[0])
        for j in range(len(cols)):
            v = jnp.where(idx == float(j), cols[j], v)
        return v

    w1 = pick(e1, pcol)
    w2 = pick(e2, pcol)
    s = w1 + w2
    w1_ref[...] = w1 / s
    w2_ref[...] = w2 / s

    lif = li.astype(jnp.float32)
    cnt = (jnp.where(lif == e1, 1.0, 0.0)
           + jnp.where(lif == e2, 1.0, 0.0))
    rows = jax.lax.broadcasted_iota(jnp.int32, (T, T), 0)
    cols_ = jax.lax.broadcasted_iota(jnp.int32, (T, T), 1)
    ltri = jnp.where(rows >= cols_, 1.0, 0.0)
    incl = jnp.dot(ltri, cnt, preferred_element_type=jnp.float32)
    ex_cum = incl - cnt

    rank0 = jnp.sum(jnp.where(lif == e1, ex_cum, 0.0), axis=1, keepdims=True)
    rank1 = jnp.sum(jnp.where(lif == e2, ex_cum, 0.0), axis=1, keepdims=True)
    d0_ref[...] = (e1 * float(C) + rank0).astype(jnp.int32)
    d1_ref[...] = (e2 * float(C) + rank1).astype(jnp.int32)
    counts = jnp.sum(cnt, axis=0, keepdims=True)
    cnt_ref[...] = counts.astype(jnp.int32)

    ccol = [jnp.sum(jnp.where(li[:1] == e, counts, 0.0), axis=1,
                    keepdims=True) for e in range(E)]
    nbcol = [jnp.floor((c + float(SB - 1)) / float(SB)) for c in ccol]
    c_incl = []
    run = jnp.zeros_like(nbcol[0])
    for e in range(E):
        run = run + nbcol[e]
        c_incl.append(run)
    svec = jax.lax.broadcasted_iota(jnp.int32, (NB, 1), 0).astype(jnp.float32)
    be = jnp.zeros((NB, 1), jnp.float32)
    for e in range(E):
        be = be + jnp.where(svec >= c_incl[e], 1.0, 0.0)
    cum_ex = [c_incl[e] - nbcol[e] for e in range(E)]
    base = jnp.zeros((NB, 1), jnp.float32)
    for e in range(E):
        base = jnp.where(be == float(e), cum_ex[e], base)
    binner = svec - base
    inactive = svec >= c_incl[E - 1]
    be = jnp.where(inactive, float(E - 1), be)
    brow = jnp.where(inactive, float(E * CBS - 1), be * float(CBS) + binner)
    bexp_ref[...] = be.astype(jnp.int32)
    brow_ref[...] = brow.astype(jnp.int32)


def _router(x, gate_w, bias):
    return pl.pallas_call(
        _router_body,
        out_shape=(
            jax.ShapeDtypeStruct((T, 1), jnp.int32),
            jax.ShapeDtypeStruct((T, 1), jnp.int32),
            jax.ShapeDtypeStruct((T, 1), jnp.float32),
            jax.ShapeDtypeStruct((T, 1), jnp.float32),
            jax.ShapeDtypeStruct((1, E), jnp.int32),
            jax.ShapeDtypeStruct((NB, 1), jnp.int32),
            jax.ShapeDtypeStruct((NB, 1), jnp.int32),
        ),
    )(x, gate_w, bias.reshape(1, E))


def _disp_body(x_hbm, d0_hbm, d1_hbm, xs_hbm, i0_v, i1_v, rows_v, sem):
    wid = lax.axis_index("s") * _NC + lax.axis_index("c")
    base = wid * TPW
    pltpu.sync_copy(d0_hbm.at[pl.ds(base, TPW)], i0_v)
    pltpu.sync_copy(d1_hbm.at[pl.ds(base, TPW)], i1_v)
    pltpu.sync_copy(x_hbm.at[pl.ds(base, TPW)], rows_v)
    c0 = pltpu.async_copy(rows_v, xs_hbm.at[i0_v], sem)
    c1 = pltpu.async_copy(rows_v, xs_hbm.at[i1_v], sem)
    c0.wait()
    c1.wait()


def _dispatch(x, d0, d1):
    mesh = plsc.VectorSubcoreMesh(core_axis_name="c", subcore_axis_name="s")
    return pl.kernel(
        _disp_body,
        out_type=jax.ShapeDtypeStruct((SLOTS, D), jnp.float32),
        mesh=mesh,
        scratch_types=[
            pltpu.VMEM((TPW,), jnp.int32),
            pltpu.VMEM((TPW,), jnp.int32),
            pltpu.VMEM((TPW, D), jnp.float32),
            pltpu.SemaphoreType.DMA,
        ],
    )(x, d0, d1)


def _mlp_body(cnt_ref, bexp_ref, brow_ref, xs_ref, wgu_ref, wd_ref, ys_ref):
    s = pl.program_id(0)
    e = bexp_ref[s]
    b_inner = brow_ref[s] - e * CBS

    @pl.when(b_inner * SB < cnt_ref[e])
    def _():
        gu = _dot_t(xs_ref[...], wgu_ref[0])
        act = jax.nn.silu(gu[:, :I]) * gu[:, I:]
        ys_ref[...] = _dot_t(act, wd_ref[0])


def _grouped_mlp(counts, bexp, brow, xs, w_gate_up, w_down):
    grid_spec = pltpu.PrefetchScalarGridSpec(
        num_scalar_prefetch=3,
        grid=(NB,),
        in_specs=[
            pl.BlockSpec((SB, D), lambda s, c, be, br: (br[s], 0)),
            pl.BlockSpec((1, 2 * I, D), lambda s, c, be, br: (be[s], 0, 0)),
            pl.BlockSpec((1, D, I), lambda s, c, be, br: (be[s], 0, 0)),
        ],
        out_specs=pl.BlockSpec((SB, D), lambda s, c, be, br: (br[s], 0)),
    )
    return pl.pallas_call(
        _mlp_body,
        grid_spec=grid_spec,
        out_shape=jax.ShapeDtypeStruct((SLOTS, D), jnp.float32),
    )(counts, bexp, brow, xs, w_gate_up, w_down)


HW = TPW // 2


def _gath_body(ys_hbm, d0_hbm, d1_hbm, y0_hbm, y1_hbm,
               ia, ib, ic, id_, b0, b1, b2, s0, s1, s2):
    wid = lax.axis_index("s") * _NC + lax.axis_index("c")
    base = wid * TPW
    pltpu.sync_copy(d0_hbm.at[pl.ds(base, HW)], ia)
    pltpu.sync_copy(d0_hbm.at[pl.ds(base + HW, HW)], ib)
    pltpu.sync_copy(d1_hbm.at[pl.ds(base, HW)], ic)
    pltpu.sync_copy(d1_hbm.at[pl.ds(base + HW, HW)], id_)
    c0 = pltpu.async_copy(ys_hbm.at[ia], b0, s0)
    c1 = pltpu.async_copy(ys_hbm.at[ib], b1, s1)
    c0.wait()
    c2 = pltpu.async_copy(ys_hbm.at[ic], b2, s2)
    pltpu.sync_copy(b0, y0_hbm.at[pl.ds(base, HW)])
    c1.wait()
    c3 = pltpu.async_copy(ys_hbm.at[id_], b0, s0)
    pltpu.sync_copy(b1, y0_hbm.at[pl.ds(base + HW, HW)])
    c2.wait()
    pltpu.sync_copy(b2, y1_hbm.at[pl.ds(base, HW)])
    c3.wait()
    pltpu.sync_copy(b0, y1_hbm.at[pl.ds(base + HW, HW)])


def _combine_gather(ys, d0, d1):
    mesh = plsc.VectorSubcoreMesh(core_axis_name="c", subcore_axis_name="s")
    return pl.kernel(
        _gath_body,
        out_type=(
            jax.ShapeDtypeStruct((T, D), jnp.float32),
            jax.ShapeDtypeStruct((T, D), jnp.float32),
        ),
        mesh=mesh,
        scratch_types=[
            pltpu.VMEM((HW,), jnp.int32),
            pltpu.VMEM((HW,), jnp.int32),
            pltpu.VMEM((HW,), jnp.int32),
            pltpu.VMEM((HW,), jnp.int32),
            pltpu.VMEM((HW, D), jnp.float32),
            pltpu.VMEM((HW, D), jnp.float32),
            pltpu.VMEM((HW, D), jnp.float32),
            pltpu.SemaphoreType.DMA,
            pltpu.SemaphoreType.DMA,
            pltpu.SemaphoreType.DMA,
        ],
    )(ys, d0, d1)


def _shared_body(x_ref, sgu_ref, sd_ref, out_ref):
    xb = x_ref[...]
    gu = _dot_t(xb, sgu_ref[...])
    h = I * NSH
    act = jax.nn.silu(gu[:, :h]) * gu[:, h:]
    out_ref[...] = _dot_t(act, sd_ref[...])


def _shared_mlp(x, shared_gate_up, shared_down):
    return pl.pallas_call(
        _shared_body,
        grid=(NT,),
        in_specs=[
            pl.BlockSpec((TB, D), lambda t: (t, 0)),
            pl.BlockSpec((2 * I * NSH, D), lambda t: (0, 0)),
            pl.BlockSpec((D, I * NSH), lambda t: (0, 0)),
        ],
        out_specs=pl.BlockSpec((TB, D), lambda t: (t, 0)),
        out_shape=jax.ShapeDtypeStruct((T, D), jnp.float32),
    )(x, shared_gate_up, shared_down)


def _final_body(x_ref, y0_ref, y1_ref, w1_ref, w2_ref, sgu_ref, sd_ref,
                out_ref):
    xb = x_ref[...]
    gu = _dot_t(xb, sgu_ref[...])
    h = I * NSH
    act = jax.nn.silu(gu[:, :h]) * gu[:, h:]
    sh = _dot_t(act, sd_ref[...])
    routed = y0_ref[...] * w1_ref[...] + y1_ref[...] * w2_ref[...]
    out_ref[...] = routed * SCALE + sh


def _shared_final(x, y0, y1, w1, w2, shared_gate_up, shared_down):
    return pl.pallas_call(
        _final_body,
        grid=(NT,),
        in_specs=[
            pl.BlockSpec((TB, D), lambda t: (t, 0)),
            pl.BlockSpec((TB, D), lambda t: (t, 0)),
            pl.BlockSpec((TB, D), lambda t: (t, 0)),
            pl.BlockSpec((TB, 1), lambda t: (t, 0)),
            pl.BlockSpec((TB, 1), lambda t: (t, 0)),
            pl.BlockSpec((2 * I * NSH, D), lambda t: (0, 0)),
            pl.BlockSpec((D, I * NSH), lambda t: (0, 0)),
        ],
        out_specs=pl.BlockSpec((TB, D), lambda t: (t, 0)),
        out_shape=jax.ShapeDtypeStruct((T, D), jnp.float32),
    )(x, y0, y1, w1, w2, shared_gate_up, shared_down)


def kernel(hidden_states, gate_w, e_score_correction_bias, w_gate_up, w_down,
           shared_gate_up, shared_down):
    x = hidden_states
    d0, d1, w1, w2, counts, bexp, brow = _router(
        x, gate_w, e_score_correction_bias)
    xs = _dispatch(x, d0.reshape(T), d1.reshape(T))
    ys = _grouped_mlp(counts.reshape(E), bexp.reshape(NB), brow.reshape(NB),
                      xs, w_gate_up, w_down)
    y0, y1 = _combine_gather(ys, d0.reshape(T), d1.reshape(T))
    return _shared_final(x, y0, y1, w1, w2, shared_gate_up, shared_down)


# speedup vs baseline: 1.0108x; 1.0108x over previous
import functools

import jax
import jax.numpy as jnp
from jax import lax
from jax.experimental import pallas as pl
from jax.experimental.pallas import tpu as pltpu
from jax.experimental.pallas import tpu_sc as plsc

T = 2048
D = 1024
E = 8
NGROUP = 4
I = 512
NSH = 2
SCALE = 1.5

TB = 256
NT = T // TB
C = T
CB = C // TB
SLOTS = E * C
SB = 512
CBS = C // SB
NB = 16

_NC, _NS = 2, 16
_NW = _NC * _NS
TPW = T // _NW


def _dot_t(a, b):
    return jax.lax.dot_general(a, b, (((1,), (1,)), ((), ())),
                               preferred_element_type=jnp.float32)


def _router_body(x_ref, gw_ref, bias_ref, d0_ref, d1_ref, w1_ref,
                 w2_ref, cnt_ref, bexp_ref, brow_ref):
    x = x_ref[...]
    logits = _dot_t(x, gw_ref[...])
    m = jnp.max(logits, axis=1, keepdims=True)
    ex = jnp.exp(logits - m)
    p = ex / jnp.sum(ex, axis=1, keepdims=True)

    li = jax.lax.broadcasted_iota(jnp.int32, (T, E), 1)
    pcol = [jnp.sum(jnp.where(li == e, p, 0.0), axis=1, keepdims=True)
            for e in range(E)]
    sccol = [pcol[e] + bias_ref[0, e] for e in range(E)]

    gs = [sccol[2 * g] + sccol[2 * g + 1] for g in range(NGROUP)]

    neg = jnp.float32(-jnp.inf)

    def first_argmax(cols):
        v = cols[0]
        for c in cols[1:]:
            v = jnp.maximum(v, c)
        idx = jnp.full_like(v, float(len(cols)))
        for j in range(len(cols) - 1, -1, -1):
            idx = jnp.where(cols[j] == v, float(j), idx)
        return v, idx

    _, g1 = first_argmax(gs)
    gs2 = [jnp.where(g1 == float(g), neg, gs[g]) for g in range(NGROUP)]
    _, g2 = first_argmax(gs2)

    masked = [jnp.where((g1 == float(e // 2)) | (g2 == float(e // 2)),
                        sccol[e], neg) for e in range(E)]
    _, e1 = first_argmax(masked)
    masked2 = [jnp.where(e1 == float(e), neg, masked[e]) for e in range(E)]
    _, e2 = first_argmax(masked2)

    def pick(idx, cols):
        v = jnp.zeros_like(cols[0])
        for j in range(len(cols)):
            v = jnp.where(idx == float(j), cols[j], v)
        return v

    w1 = pick(e1, pcol)
    w2 = pick(e2, pcol)
    s = w1 + w2
    w1_ref[...] = w1 / s
    w2_ref[...] = w2 / s

    lif = li.astype(jnp.float32)
    cnt = (jnp.where(lif == e1, 1.0, 0.0)
           + jnp.where(lif == e2, 1.0, 0.0))
    rows = jax.lax.broadcasted_iota(jnp.int32, (T, T), 0)
    cols_ = jax.lax.broadcasted_iota(jnp.int32, (T, T), 1)
    ltri = jnp.where(rows >= cols_, 1.0, 0.0)
    incl = jnp.dot(ltri, cnt, preferred_element_type=jnp.float32)
    ex_cum = incl - cnt

    rank0 = jnp.sum(jnp.where(lif == e1, ex_cum, 0.0), axis=1, keepdims=True)
    rank1 = jnp.sum(jnp.where(lif == e2, ex_cum, 0.0), axis=1, keepdims=True)
    d0_ref[...] = (e1 * float(C) + rank0).astype(jnp.int32)
    d1_ref[...] = (e2 * float(C) + rank1).astype(jnp.int32)
    counts = jnp.sum(cnt, axis=0, keepdims=True)
    cnt_ref[...] = counts.astype(jnp.int32)

    ccol = [jnp.sum(jnp.where(li[:1] == e, counts, 0.0), axis=1,
                    keepdims=True) for e in range(E)]
    nbcol = [jnp.floor((c + float(SB - 1)) / float(SB)) for c in ccol]
    c_incl = []
    run = jnp.zeros_like(nbcol[0])
    for e in range(E):
        run = run + nbcol[e]
        c_incl.append(run)
    svec = jax.lax.broadcasted_iota(jnp.int32, (NB, 1), 0).astype(jnp.float32)
    be = jnp.zeros((NB, 1), jnp.float32)
    for e in range(E):
        be = be + jnp.where(svec >= c_incl[e], 1.0, 0.0)
    cum_ex = [c_incl[e] - nbcol[e] for e in range(E)]
    base = jnp.zeros((NB, 1), jnp.float32)
    for e in range(E):
        base = jnp.where(be == float(e), cum_ex[e], base)
    binner = svec - base
    inactive = svec >= c_incl[E - 1]
    be = jnp.where(inactive, float(E - 1), be)
    brow = jnp.where(inactive, float(E * CBS - 1), be * float(CBS) + binner)
    bexp_ref[...] = be.astype(jnp.int32)
    brow_ref[...] = brow.astype(jnp.int32)


def _router(x, gate_w, bias):
    return pl.pallas_call(
        _router_body,
        out_shape=(
            jax.ShapeDtypeStruct((T, 1), jnp.int32),
            jax.ShapeDtypeStruct((T, 1), jnp.int32),
            jax.ShapeDtypeStruct((T, 1), jnp.float32),
            jax.ShapeDtypeStruct((T, 1), jnp.float32),
            jax.ShapeDtypeStruct((1, E), jnp.int32),
            jax.ShapeDtypeStruct((NB, 1), jnp.int32),
            jax.ShapeDtypeStruct((NB, 1), jnp.int32),
        ),
    )(x, gate_w, bias.reshape(1, E))


def _disp_body(x_hbm, d0_hbm, d1_hbm, xs_hbm, i0_v, i1_v, rows_v, sem):
    wid = lax.axis_index("s") * _NC + lax.axis_index("c")
    base = wid * TPW
    pltpu.sync_copy(d0_hbm.at[pl.ds(base, TPW)], i0_v)
    pltpu.sync_copy(d1_hbm.at[pl.ds(base, TPW)], i1_v)
    pltpu.sync_copy(x_hbm.at[pl.ds(base, TPW)], rows_v)
    c0 = pltpu.async_copy(rows_v, xs_hbm.at[i0_v], sem)
    c1 = pltpu.async_copy(rows_v, xs_hbm.at[i1_v], sem)
    c0.wait()
    c1.wait()


def _dispatch(x, d0, d1):
    mesh = plsc.VectorSubcoreMesh(core_axis_name="c", subcore_axis_name="s")
    return pl.kernel(
        _disp_body,
        out_type=jax.ShapeDtypeStruct((SLOTS, D), jnp.float32),
        mesh=mesh,
        scratch_types=[
            pltpu.VMEM((TPW,), jnp.int32),
            pltpu.VMEM((TPW,), jnp.int32),
            pltpu.VMEM((TPW, D), jnp.float32),
            pltpu.SemaphoreType.DMA,
        ],
    )(x, d0, d1)


def _mlp_body(cnt_ref, bexp_ref, brow_ref, xs_ref, wgu_ref, wd_ref, ys_ref):
    s = pl.program_id(0)
    e = bexp_ref[s]
    b_inner = brow_ref[s] - e * CBS

    @pl.when(b_inner * SB < cnt_ref[e])
    def _():
        gu = _dot_t(xs_ref[...], wgu_ref[0])
        act = jax.nn.silu(gu[:, :I]) * gu[:, I:]
        ys_ref[...] = _dot_t(act, wd_ref[0])


def _grouped_mlp(counts, bexp, brow, xs, w_gate_up, w_down):
    grid_spec = pltpu.PrefetchScalarGridSpec(
        num_scalar_prefetch=3,
        grid=(NB,),
        in_specs=[
            pl.BlockSpec((SB, D), lambda s, c, be, br: (br[s], 0)),
            pl.BlockSpec((1, 2 * I, D), lambda s, c, be, br: (be[s], 0, 0)),
            pl.BlockSpec((1, D, I), lambda s, c, be, br: (be[s], 0, 0)),
        ],
        out_specs=pl.BlockSpec((SB, D), lambda s, c, be, br: (br[s], 0)),
    )
    return pl.pallas_call(
        _mlp_body,
        grid_spec=grid_spec,
        out_shape=jax.ShapeDtypeStruct((SLOTS, D), jnp.float32),
    )(counts, bexp, brow, xs, w_gate_up, w_down)


HW = TPW // 2


def _gath_body(ys_hbm, d0_hbm, d1_hbm, y0_hbm, y1_hbm,
               ia, ib, ic, id_, b0, b1, b2, s0, s1, s2):
    wid = lax.axis_index("s") * _NC + lax.axis_index("c")
    base = wid * TPW
    pltpu.sync_copy(d0_hbm.at[pl.ds(base, HW)], ia)
    pltpu.sync_copy(d0_hbm.at[pl.ds(base + HW, HW)], ib)
    pltpu.sync_copy(d1_hbm.at[pl.ds(base, HW)], ic)
    pltpu.sync_copy(d1_hbm.at[pl.ds(base + HW, HW)], id_)
    c0 = pltpu.async_copy(ys_hbm.at[ia], b0, s0)
    c1 = pltpu.async_copy(ys_hbm.at[ib], b1, s1)
    c0.wait()
    c2 = pltpu.async_copy(ys_hbm.at[ic], b2, s2)
    pltpu.sync_copy(b0, y0_hbm.at[pl.ds(base, HW)])
    c1.wait()
    c3 = pltpu.async_copy(ys_hbm.at[id_], b0, s0)
    pltpu.sync_copy(b1, y0_hbm.at[pl.ds(base + HW, HW)])
    c2.wait()
    pltpu.sync_copy(b2, y1_hbm.at[pl.ds(base, HW)])
    c3.wait()
    pltpu.sync_copy(b0, y1_hbm.at[pl.ds(base + HW, HW)])


def _combine_gather(ys, d0, d1):
    mesh = plsc.VectorSubcoreMesh(core_axis_name="c", subcore_axis_name="s")
    return pl.kernel(
        _gath_body,
        out_type=(
            jax.ShapeDtypeStruct((T, D), jnp.float32),
            jax.ShapeDtypeStruct((T, D), jnp.float32),
        ),
        mesh=mesh,
        scratch_types=[
            pltpu.VMEM((HW,), jnp.int32),
            pltpu.VMEM((HW,), jnp.int32),
            pltpu.VMEM((HW,), jnp.int32),
            pltpu.VMEM((HW,), jnp.int32),
            pltpu.VMEM((HW, D), jnp.float32),
            pltpu.VMEM((HW, D), jnp.float32),
            pltpu.VMEM((HW, D), jnp.float32),
            pltpu.SemaphoreType.DMA,
            pltpu.SemaphoreType.DMA,
            pltpu.SemaphoreType.DMA,
        ],
    )(ys, d0, d1)


def _shared_body(x_ref, sgu_ref, sd_ref, out_ref):
    xb = x_ref[...]
    gu = _dot_t(xb, sgu_ref[...])
    h = I * NSH
    act = jax.nn.silu(gu[:, :h]) * gu[:, h:]
    out_ref[...] = _dot_t(act, sd_ref[...])


def _shared_mlp(x, shared_gate_up, shared_down):
    return pl.pallas_call(
        _shared_body,
        grid=(NT,),
        in_specs=[
            pl.BlockSpec((TB, D), lambda t: (t, 0)),
            pl.BlockSpec((2 * I * NSH, D), lambda t: (0, 0)),
            pl.BlockSpec((D, I * NSH), lambda t: (0, 0)),
        ],
        out_specs=pl.BlockSpec((TB, D), lambda t: (t, 0)),
        out_shape=jax.ShapeDtypeStruct((T, D), jnp.float32),
    )(x, shared_gate_up, shared_down)


def _final_body(y0_ref, y1_ref, w1_ref, w2_ref, sh_ref, out_ref):
    routed = y0_ref[...] * w1_ref[...] + y1_ref[...] * w2_ref[...]
    out_ref[...] = routed * SCALE + sh_ref[...]


def _final_combine(y0, y1, w1, w2, sh):
    return pl.pallas_call(
        _final_body,
        grid=(NT,),
        in_specs=[
            pl.BlockSpec((TB, D), lambda t: (t, 0)),
            pl.BlockSpec((TB, D), lambda t: (t, 0)),
            pl.BlockSpec((TB, 1), lambda t: (t, 0)),
            pl.BlockSpec((TB, 1), lambda t: (t, 0)),
            pl.BlockSpec((TB, D), lambda t: (t, 0)),
        ],
        out_specs=pl.BlockSpec((TB, D), lambda t: (t, 0)),
        out_shape=jax.ShapeDtypeStruct((T, D), jnp.float32),
    )(y0, y1, w1, w2, sh)


def kernel(hidden_states, gate_w, e_score_correction_bias, w_gate_up, w_down,
           shared_gate_up, shared_down):
    x = hidden_states
    d0, d1, w1, w2, counts, bexp, brow = _router(
        x, gate_w, e_score_correction_bias)
    xs = _dispatch(x, d0.reshape(T), d1.reshape(T))
    sh = _shared_mlp(x, shared_gate_up, shared_down)
    ys = _grouped_mlp(counts.reshape(E), bexp.reshape(NB), brow.reshape(NB),
                      xs, w_gate_up, w_down)
    y0, y1 = _combine_gather(ys, d0.reshape(T), d1.reshape(T))
    return _final_combine(y0, y1, w1, w2, sh)
